# Initial kernel scaffold; baseline (speedup 1.0000x reference)
import jax
import jax.numpy as jnp
from jax.experimental import pallas as pl


def kernel(input, target):
    raise NotImplementedError("write your pallas kernel here")


# speedup vs baseline: 15.4589x; 15.4589x over previous
import jax
import jax.numpy as jnp
from jax import lax
from jax.experimental import pallas as pl
from jax.experimental.pallas import tpu as pltpu
from jax.experimental.pallas import tpu_sc as plsc

B = 16
N = 512 * 512
K = 32768
EMAX = 8.0
SCALE = (K - 1) / EMAX
NTILES = 32
HALF = N // 2
CH = 8192
KR, KC = 256, 128


def _sc_hist(x_hbm, t_hbm, out_hbm, xbuf, tbuf, hn, hs, hw):
    cid = lax.axis_index("c")
    sid = lax.axis_index("s")
    wid = sid * 2 + cid
    base = wid * HALF

    zeros = jnp.zeros((16,), jnp.float32)

    def zbody(i, carry):
        hn[pl.ds(i * 16, 16)] = zeros
        hs[pl.ds(i * 16, 16)] = zeros
        hw[pl.ds(i * 16, 16)] = zeros
        return carry

    lax.fori_loop(0, K // 16, zbody, 0)

    ones = jnp.ones((16,), jnp.float32)

    def body(i, carry):
        xv = xbuf[pl.ds(i * 16, 16)]
        tv = tbuf[pl.ds(i * 16, 16)]
        e = 1.0 - xv * (2.0 * tv - 1.0)
        f = e * SCALE
        fc = jnp.minimum(jnp.maximum(f, 0.0), float(K - 2))
        fi = fc.astype(jnp.int32) + 1
        idx = jnp.where(f > 0.0, fi, 0)
        plsc.addupdate_scatter(hn, [idx], ones)
        plsc.addupdate_scatter(hs, [idx], tv)
        plsc.addupdate_scatter(hw, [idx], jnp.maximum(e, 0.0))
        return carry

    for c in range(HALF // CH):
        pltpu.sync_copy(x_hbm.at[pl.ds(base + c * CH, CH)], xbuf)
        pltpu.sync_copy(t_hbm.at[pl.ds(base + c * CH, CH)], tbuf)
        lax.fori_loop(0, CH // 16, body, 0)

    obase = wid * 3 * K
    pltpu.sync_copy(hn, out_hbm.at[pl.ds(obase, K)])
    pltpu.sync_copy(hs, out_hbm.at[pl.ds(obase + K, K)])
    pltpu.sync_copy(hw, out_hbm.at[pl.ds(obase + 2 * K, K)])


_phase1 = pl.kernel(
    _sc_hist,
    out_type=jax.ShapeDtypeStruct((NTILES * 3 * K,), jnp.float32),
    mesh=plsc.VectorSubcoreMesh(core_axis_name="c", subcore_axis_name="s"),
    compiler_params=pltpu.CompilerParams(needs_layout_passes=False),
    scratch_types=[
        pltpu.VMEM((CH,), jnp.float32),
        pltpu.VMEM((CH,), jnp.float32),
        pltpu.VMEM((K,), jnp.float32),
        pltpu.VMEM((K,), jnp.float32),
        pltpu.VMEM((K,), jnp.float32),
    ],
)


def _tc_finish(h_ref, o_ref):
    i = pl.program_id(0)
    n2 = h_ref[0, 0, 0] + h_ref[0, 1, 0]
    s2 = h_ref[0, 0, 1] + h_ref[0, 1, 1]
    w2 = h_ref[0, 0, 2] + h_ref[0, 1, 2]

    r = lax.broadcasted_iota(jnp.int32, (KC, KC), 0)
    c = lax.broadcasted_iota(jnp.int32, (KC, KC), 1)
    upper = (r <= c).astype(jnp.float32)
    rr = lax.broadcasted_iota(jnp.int32, (KR, KR), 0)
    rc = lax.broadcasted_iota(jnp.int32, (KR, KR), 1)
    lstrict = (rc < rr).astype(jnp.float32)

    incl_n = jnp.dot(n2, upper, preferred_element_type=jnp.float32)
    incl_s = jnp.dot(s2, upper, preferred_element_type=jnp.float32)
    rt_n = jnp.sum(n2, axis=1, keepdims=True)
    rt_s = jnp.sum(s2, axis=1, keepdims=True)
    incl_n = incl_n + jnp.dot(lstrict, rt_n, preferred_element_type=jnp.float32)
    incl_s = incl_s + jnp.dot(lstrict, rt_s, preferred_element_type=jnp.float32)

    tot_n = jnp.sum(n2)
    g = jnp.sum(s2)
    m_above = tot_n - incl_n
    s_above = g - incl_s
    d1 = g + m_above - s_above
    d2 = d1 + n2 - s2
    num = n2 * (g - s_above) + m_above * s2
    dj = jnp.where(
        d1 > 0.0,
        num / jnp.maximum(d1 * d2, 1.0),
        (m_above + n2) / jnp.maximum(d2, 1.0),
    )
    loss = jnp.sum(w2 * dj / jnp.maximum(n2, 1.0))

    @pl.when(i == 0)
    def _():
        o_ref[0, 0] = 0.0

    o_ref[0, 0] += loss * (1.0 / B)


def _phase2(hist):
    return pl.pallas_call(
        _tc_finish,
        grid=(B,),
        in_specs=[
            pl.BlockSpec((1, 2, 3, KR, KC), lambda i: (i, 0, 0, 0, 0)),
        ],
        out_specs=pl.BlockSpec(
            (1, 1), lambda i: (0, 0), memory_space=pltpu.SMEM
        ),
        out_shape=jax.ShapeDtypeStruct((1, 1), jnp.float32),
    )(hist)


def kernel(input, target):
    x2 = input.reshape(B * N)
    t2 = target.reshape(B * N)
    hist = _phase1(x2, t2)
    h5 = hist.reshape(B, 2, 3, KR, KC)
    out = _phase2(h5)
    return out[0, 0]


# speedup vs baseline: 40.8855x; 2.6448x over previous
import jax
import jax.numpy as jnp
from jax import lax
from jax.experimental import pallas as pl
from jax.experimental.pallas import tpu as pltpu
from jax.experimental.pallas import tpu_sc as plsc

B = 16
N = 512 * 512
K = 16384
EMAX = 8.0
SCALE = (K - 1) / EMAX
NTILES = 32
ROWS_PER_TILE = 256
CROWS = 16
NCH = ROWS_PER_TILE // CROWS
VECS = CROWS * 512 // 16
KR, KC = 128, 128


def _sc_hist(x_hbm, t_hbm, out_hbm, xb0, xb1, tb0, tb1, hn, hs, hw,
             sx0, sx1, st0, st1):
    cid = lax.axis_index("c")
    sid = lax.axis_index("s")
    wid = sid * 2 + cid
    img = wid // 2
    r0 = (wid % 2) * ROWS_PER_TILE

    zeros = jnp.zeros((16,), jnp.float32)

    @plsc.parallel_loop(0, K // 16, unroll=8)
    def _(i):
        hn[pl.ds(i * 16, 16)] = zeros
        hs[pl.ds(i * 16, 16)] = zeros
        hw[pl.ds(i * 16, 16)] = zeros

    ones = jnp.ones((16,), jnp.float32)
    xbufs, tbufs = (xb0, xb1), (tb0, tb1)
    sxs, sts = (sx0, sx1), (st0, st1)

    def start(c):
        rr = pl.ds(r0 + c * CROWS, CROWS)
        hx = pltpu.async_copy(x_hbm.at[img, 0, rr, :], xbufs[c % 2], sxs[c % 2])
        ht = pltpu.async_copy(t_hbm.at[img, 0, rr, :], tbufs[c % 2], sts[c % 2])
        return hx, ht

    handles = start(0)
    for c in range(NCH):
        prev = handles
        if c + 1 < NCH:
            handles = start(c + 1)
        prev[0].wait()
        prev[1].wait()
        xbuf, tbuf = xbufs[c % 2], tbufs[c % 2]

        @plsc.parallel_loop(0, VECS, unroll=4)
        def _(i):
            r = lax.shift_right_logical(i, 5)
            k = jnp.bitwise_and(i, 31)
            xv = xbuf[r, pl.ds(k * 16, 16)]
            tv = tbuf[r, pl.ds(k * 16, 16)]
            e = 1.0 - xv * (2.0 * tv - 1.0)
            f = e * SCALE
            fc = jnp.minimum(jnp.maximum(f, 0.0), float(K - 2))
            fi = fc.astype(jnp.int32) + 1
            idx = jnp.where(f > 0.0, fi, 0)
            plsc.addupdate_scatter(hn, [idx], ones)
            plsc.addupdate_scatter(hs, [idx], tv)
            plsc.addupdate_scatter(hw, [idx], jnp.maximum(e, 0.0))

    obase = wid * 3 * K
    pltpu.sync_copy(hn, out_hbm.at[pl.ds(obase, K)])
    pltpu.sync_copy(hs, out_hbm.at[pl.ds(obase + K, K)])
    pltpu.sync_copy(hw, out_hbm.at[pl.ds(obase + 2 * K, K)])


_phase1 = pl.kernel(
    _sc_hist,
    out_type=jax.ShapeDtypeStruct((NTILES * 3 * K,), jnp.float32),
    mesh=plsc.VectorSubcoreMesh(core_axis_name="c", subcore_axis_name="s"),
    compiler_params=pltpu.CompilerParams(needs_layout_passes=False),
    scratch_types=[
        pltpu.VMEM((CROWS, 512), jnp.float32),
        pltpu.VMEM((CROWS, 512), jnp.float32),
        pltpu.VMEM((CROWS, 512), jnp.float32),
        pltpu.VMEM((CROWS, 512), jnp.float32),
        pltpu.VMEM((K,), jnp.float32),
        pltpu.VMEM((K,), jnp.float32),
        pltpu.VMEM((K,), jnp.float32),
        pltpu.SemaphoreType.DMA,
        pltpu.SemaphoreType.DMA,
        pltpu.SemaphoreType.DMA,
        pltpu.SemaphoreType.DMA,
    ],
)


def _tc_finish(h_ref, o_ref):
    i = pl.program_id(0)
    n2 = h_ref[0, 0, 0] + h_ref[0, 1, 0]
    s2 = h_ref[0, 0, 1] + h_ref[0, 1, 1]
    w2 = h_ref[0, 0, 2] + h_ref[0, 1, 2]

    r = lax.broadcasted_iota(jnp.int32, (KC, KC), 0)
    c = lax.broadcasted_iota(jnp.int32, (KC, KC), 1)
    upper = (r <= c).astype(jnp.float32)
    rr = lax.broadcasted_iota(jnp.int32, (KR, KR), 0)
    rc = lax.broadcasted_iota(jnp.int32, (KR, KR), 1)
    lstrict = (rc < rr).astype(jnp.float32)

    incl_n = jnp.dot(n2, upper, preferred_element_type=jnp.float32)
    incl_s = jnp.dot(s2, upper, preferred_element_type=jnp.float32)
    rt_n = jnp.sum(n2, axis=1, keepdims=True)
    rt_s = jnp.sum(s2, axis=1, keepdims=True)
    incl_n = incl_n + jnp.dot(lstrict, rt_n, preferred_element_type=jnp.float32)
    incl_s = incl_s + jnp.dot(lstrict, rt_s, preferred_element_type=jnp.float32)

    tot_n = jnp.sum(n2)
    g = jnp.sum(s2)
    m_above = tot_n - incl_n
    s_above = g - incl_s
    d1 = g + m_above - s_above
    d2 = d1 + n2 - s2
    num = n2 * (g - s_above) + m_above * s2
    dj = jnp.where(
        d1 > 0.0,
        num / jnp.maximum(d1 * d2, 1.0),
        (m_above + n2) / jnp.maximum(d2, 1.0),
    )
    loss = jnp.sum(w2 * dj / jnp.maximum(n2, 1.0))

    @pl.when(i == 0)
    def _():
        o_ref[0, 0] = 0.0

    o_ref[0, 0] += loss * (1.0 / B)


def _phase2(hist):
    return pl.pallas_call(
        _tc_finish,
        grid=(B,),
        in_specs=[
            pl.BlockSpec((1, 2, 3, KR, KC), lambda i: (i, 0, 0, 0, 0)),
        ],
        out_specs=pl.BlockSpec(
            (1, 1), lambda i: (0, 0), memory_space=pltpu.SMEM
        ),
        out_shape=jax.ShapeDtypeStruct((1, 1), jnp.float32),
    )(hist)


def kernel(input, target):
    hist = _phase1(input, target)
    h5 = hist.reshape(B, 2, 3, KR, KC)
    out = _phase2(h5)
    return out[0, 0]


# speedup vs baseline: 61.4136x; 1.5021x over previous
import jax
import jax.numpy as jnp
from jax import lax
from jax.experimental import pallas as pl
from jax.experimental.pallas import tpu as pltpu
from jax.experimental.pallas import tpu_sc as plsc

B = 16
N = 512 * 512
K = 16384
EMAX = 8.0
SCALE = (K - 1) / EMAX
NTILES = 32
ROWS_PER_TILE = 256
CROWS = 16
NCH = ROWS_PER_TILE // CROWS
VECS = CROWS * 512 // 16
KR, KC = 128, 128


def _sc_hist(x_hbm, t_hbm, opk_hbm, ow_hbm, xb0, xb1, tb0, tb1, hp, hw,
             sx0, sx1, st0, st1):
    cid = lax.axis_index("c")
    sid = lax.axis_index("s")
    wid = sid * 2 + cid
    img = wid // 2
    r0 = (wid % 2) * ROWS_PER_TILE

    zi = jnp.zeros((16,), jnp.int32)
    zf = jnp.zeros((16,), jnp.float32)

    @plsc.parallel_loop(0, K // 16, unroll=8)
    def _(i):
        hp[pl.ds(i * 16, 16)] = zi
        hw[pl.ds(i * 16, 16)] = zf

    xbufs, tbufs = (xb0, xb1), (tb0, tb1)
    sxs, sts = (sx0, sx1), (st0, st1)

    def start(c):
        rr = pl.ds(r0 + c * CROWS, CROWS)
        hx = pltpu.async_copy(x_hbm.at[img, 0, rr, :], xbufs[c % 2], sxs[c % 2])
        ht = pltpu.async_copy(t_hbm.at[img, 0, rr, :], tbufs[c % 2], sts[c % 2])
        return hx, ht

    handles = start(0)
    for c in range(NCH):
        prev = handles
        if c + 1 < NCH:
            handles = start(c + 1)
        prev[0].wait()
        prev[1].wait()
        xbuf, tbuf = xbufs[c % 2], tbufs[c % 2]

        @plsc.parallel_loop(0, VECS, unroll=4)
        def _(i):
            r = lax.shift_right_logical(i, 5)
            k = jnp.bitwise_and(i, 31)
            xv = xbuf[r, pl.ds(k * 16, 16)]
            tv = tbuf[r, pl.ds(k * 16, 16)]
            e = 1.0 - xv * (2.0 * tv - 1.0)
            f = e * SCALE
            fc = jnp.minimum(jnp.maximum(f, 0.0), float(K - 2))
            fi = fc.astype(jnp.int32) + 1
            pos = f > 0.0
            idx = jnp.where(pos, fi, 0)
            pk = jnp.left_shift(tv.astype(jnp.int32), 16) + 1
            plsc.addupdate_scatter(hp, [idx], pk)
            plsc.addupdate_scatter(hw, [idx], jnp.maximum(e, 0.0), mask=pos)

    pltpu.sync_copy(hp, opk_hbm.at[pl.ds(wid * K, K)])
    pltpu.sync_copy(hw, ow_hbm.at[pl.ds(wid * K, K)])


_phase1 = pl.kernel(
    _sc_hist,
    out_type=(
        jax.ShapeDtypeStruct((NTILES * K,), jnp.int32),
        jax.ShapeDtypeStruct((NTILES * K,), jnp.float32),
    ),
    mesh=plsc.VectorSubcoreMesh(core_axis_name="c", subcore_axis_name="s"),
    compiler_params=pltpu.CompilerParams(needs_layout_passes=False),
    scratch_types=[
        pltpu.VMEM((CROWS, 512), jnp.float32),
        pltpu.VMEM((CROWS, 512), jnp.float32),
        pltpu.VMEM((CROWS, 512), jnp.float32),
        pltpu.VMEM((CROWS, 512), jnp.float32),
        pltpu.VMEM((K,), jnp.int32),
        pltpu.VMEM((K,), jnp.float32),
        pltpu.SemaphoreType.DMA,
        pltpu.SemaphoreType.DMA,
        pltpu.SemaphoreType.DMA,
        pltpu.SemaphoreType.DMA,
    ],
)


def _tc_finish(hp_ref, hw_ref, o_ref):
    i = pl.program_id(0)
    pk = hp_ref[0, 0] + hp_ref[0, 1]
    n2 = jnp.bitwise_and(pk, 0xFFFF).astype(jnp.float32)
    s2 = jnp.right_shift(pk, 16).astype(jnp.float32)
    w2 = hw_ref[0, 0] + hw_ref[0, 1]

    r = lax.broadcasted_iota(jnp.int32, (KC, KC), 0)
    c = lax.broadcasted_iota(jnp.int32, (KC, KC), 1)
    upper = (r <= c).astype(jnp.float32)
    rr = lax.broadcasted_iota(jnp.int32, (KR, KR), 0)
    rc = lax.broadcasted_iota(jnp.int32, (KR, KR), 1)
    lstrict = (rc < rr).astype(jnp.float32)

    incl_n = jnp.dot(n2, upper, preferred_element_type=jnp.float32)
    incl_s = jnp.dot(s2, upper, preferred_element_type=jnp.float32)
    rt_n = jnp.sum(n2, axis=1, keepdims=True)
    rt_s = jnp.sum(s2, axis=1, keepdims=True)
    incl_n = incl_n + jnp.dot(lstrict, rt_n, preferred_element_type=jnp.float32)
    incl_s = incl_s + jnp.dot(lstrict, rt_s, preferred_element_type=jnp.float32)

    tot_n = jnp.sum(n2)
    g = jnp.sum(s2)
    m_above = tot_n - incl_n
    s_above = g - incl_s
    d1 = g + m_above - s_above
    d2 = d1 + n2 - s2
    num = n2 * (g - s_above) + m_above * s2
    dj = jnp.where(
        d1 > 0.0,
        num / jnp.maximum(d1 * d2, 1.0),
        (m_above + n2) / jnp.maximum(d2, 1.0),
    )
    loss = jnp.sum(w2 * dj / jnp.maximum(n2, 1.0))

    @pl.when(i == 0)
    def _():
        o_ref[0, 0] = 0.0

    o_ref[0, 0] += loss * (1.0 / B)


def _phase2(hpk, hw):
    return pl.pallas_call(
        _tc_finish,
        grid=(B,),
        in_specs=[
            pl.BlockSpec((1, 2, KR, KC), lambda i: (i, 0, 0, 0)),
            pl.BlockSpec((1, 2, KR, KC), lambda i: (i, 0, 0, 0)),
        ],
        out_specs=pl.BlockSpec(
            (1, 1), lambda i: (0, 0), memory_space=pltpu.SMEM
        ),
        out_shape=jax.ShapeDtypeStruct((1, 1), jnp.float32),
    )(hpk, hw)


def kernel(input, target):
    hpk, hw = _phase1(input, target)
    out = _phase2(
        hpk.reshape(B, 2, KR, KC), hw.reshape(B, 2, KR, KC)
    )
    return out[0, 0]


# speedup vs baseline: 69.7906x; 1.1364x over previous
import jax
import jax.numpy as jnp
from jax import lax
from jax.experimental import pallas as pl
from jax.experimental.pallas import tpu as pltpu
from jax.experimental.pallas import tpu_sc as plsc

B = 16
N = 512 * 512
K = 16384
EMAX = 8.0
SCALE = (K - 1) / EMAX
NTILES = 32
ROWS_PER_TILE = 256
CROWS = 16
NCH = ROWS_PER_TILE // CROWS
VECS = CROWS * 512 // 16
KR, KC = 128, 128


def _sc_hist(x_hbm, t_hbm, opk_hbm, ow_hbm, xb0, xb1, tb0, tb1, hp, hw,
             sx0, sx1, st0, st1):
    cid = lax.axis_index("c")
    sid = lax.axis_index("s")
    wid = sid * 2 + cid
    img = wid // 2
    r0 = (wid % 2) * ROWS_PER_TILE

    zi = jnp.zeros((16,), jnp.int32)
    zf = jnp.zeros((16,), jnp.float32)

    @plsc.parallel_loop(0, K // 16, unroll=8)
    def _(i):
        hp[pl.ds(i * 16, 16)] = zi
        hw[pl.ds(i * 16, 16)] = zf

    xbufs, tbufs = (xb0, xb1), (tb0, tb1)
    sxs, sts = (sx0, sx1), (st0, st1)

    def start(c):
        rr = pl.ds(r0 + c * CROWS, CROWS)
        hx = pltpu.async_copy(x_hbm.at[img, 0, rr, :], xbufs[c % 2], sxs[c % 2])
        ht = pltpu.async_copy(t_hbm.at[img, 0, rr, :], tbufs[c % 2], sts[c % 2])
        return hx, ht

    handles = start(0)
    for c in range(NCH):
        prev = handles
        if c + 1 < NCH:
            handles = start(c + 1)
        prev[0].wait()
        prev[1].wait()
        xbuf, tbuf = xbufs[c % 2], tbufs[c % 2]

        @plsc.parallel_loop(0, VECS, unroll=8)
        def _(i):
            r = lax.shift_right_logical(i, 5)
            k = jnp.bitwise_and(i, 31)
            xv = xbuf[r, pl.ds(k * 16, 16)]
            tv = tbuf[r, pl.ds(k * 16, 16)]
            e = 1.0 - xv * (2.0 * tv - 1.0)
            f = e * SCALE
            fc = jnp.minimum(jnp.maximum(f, 0.0), float(K - 2))
            fi = fc.astype(jnp.int32) + 1
            pos = f > 0.0
            idx = jnp.where(pos, fi, 0)
            pk = jnp.left_shift(tv.astype(jnp.int32), 16) + 1
            plsc.addupdate_scatter(hp, [idx], pk)
            plsc.addupdate_scatter(hw, [idx], jnp.maximum(e, 0.0), mask=pos)

    pltpu.sync_copy(hp, opk_hbm.at[pl.ds(wid * K, K)])
    pltpu.sync_copy(hw, ow_hbm.at[pl.ds(wid * K, K)])


_phase1 = pl.kernel(
    _sc_hist,
    out_type=(
        jax.ShapeDtypeStruct((NTILES * K,), jnp.int32),
        jax.ShapeDtypeStruct((NTILES * K,), jnp.float32),
    ),
    mesh=plsc.VectorSubcoreMesh(core_axis_name="c", subcore_axis_name="s"),
    compiler_params=pltpu.CompilerParams(needs_layout_passes=False),
    scratch_types=[
        pltpu.VMEM((CROWS, 512), jnp.float32),
        pltpu.VMEM((CROWS, 512), jnp.float32),
        pltpu.VMEM((CROWS, 512), jnp.float32),
        pltpu.VMEM((CROWS, 512), jnp.float32),
        pltpu.VMEM((K,), jnp.int32),
        pltpu.VMEM((K,), jnp.float32),
        pltpu.SemaphoreType.DMA,
        pltpu.SemaphoreType.DMA,
        pltpu.SemaphoreType.DMA,
        pltpu.SemaphoreType.DMA,
    ],
)


def _tc_finish(hp_ref, hw_ref, o_ref):
    pk = hp_ref[:, 0] + hp_ref[:, 1]
    n3 = jnp.bitwise_and(pk, 0xFFFF).astype(jnp.float32)
    s3 = jnp.right_shift(pk, 16).astype(jnp.float32)
    w3 = hw_ref[:, 0] + hw_ref[:, 1]

    r = lax.broadcasted_iota(jnp.int32, (KC, KC), 0)
    c = lax.broadcasted_iota(jnp.int32, (KC, KC), 1)
    upper = (r <= c).astype(jnp.float32)
    strict = (r < c).astype(jnp.float32)

    incl_n = jnp.dot(n3.reshape(B * KR, KC), upper,
                     preferred_element_type=jnp.float32).reshape(B, KR, KC)
    incl_s = jnp.dot(s3.reshape(B * KR, KC), upper,
                     preferred_element_type=jnp.float32).reshape(B, KR, KC)
    rt_n = jnp.sum(n3, axis=2)
    rt_s = jnp.sum(s3, axis=2)
    prev_n = jnp.dot(rt_n, strict, preferred_element_type=jnp.float32)
    prev_s = jnp.dot(rt_s, strict, preferred_element_type=jnp.float32)
    incl_n = incl_n + prev_n[:, :, None]
    incl_s = incl_s + prev_s[:, :, None]

    tot_n = jnp.sum(rt_n, axis=1)[:, None, None]
    g = jnp.sum(rt_s, axis=1)[:, None, None]
    m_above = tot_n - incl_n
    s_above = g - incl_s
    d1 = g + m_above - s_above
    d2 = d1 + n3 - s3
    num = n3 * (g - s_above) + m_above * s3
    dj = jnp.where(
        d1 > 0.0,
        num / jnp.maximum(d1 * d2, 1.0),
        (m_above + n3) / jnp.maximum(d2, 1.0),
    )
    o_ref[0, 0] = jnp.sum(w3 * dj / jnp.maximum(n3, 1.0)) * (1.0 / B)


def _phase2(hpk, hw):
    return pl.pallas_call(
        _tc_finish,
        in_specs=[
            pl.BlockSpec((B, 2, KR, KC), lambda: (0, 0, 0, 0)),
            pl.BlockSpec((B, 2, KR, KC), lambda: (0, 0, 0, 0)),
        ],
        out_specs=pl.BlockSpec(
            (1, 1), lambda: (0, 0), memory_space=pltpu.SMEM
        ),
        out_shape=jax.ShapeDtypeStruct((1, 1), jnp.float32),
    )(hpk, hw)


def kernel(input, target):
    hpk, hw = _phase1(input, target)
    out = _phase2(
        hpk.reshape(B, 2, KR, KC), hw.reshape(B, 2, KR, KC)
    )
    return out[0, 0]


# speedup vs baseline: 78.4991x; 1.1248x over previous
import jax
import jax.numpy as jnp
from jax import lax
from jax.experimental import pallas as pl
from jax.experimental.pallas import tpu as pltpu
from jax.experimental.pallas import tpu_sc as plsc

B = 16
N = 512 * 512
K = 16384
EMAX = 8.0
SCALE = (K - 1) / EMAX
NTILES = 32
ROWS_PER_TILE = 256
CROWS = 16
NCH = ROWS_PER_TILE // CROWS
VECS = CROWS * 512 // 16
KR, KC = 128, 128


def _sc_hist(x_hbm, t_hbm, opk_hbm, xb0, xb1, tb0, tb1, hp,
             sx0, sx1, st0, st1):
    cid = lax.axis_index("c")
    sid = lax.axis_index("s")
    wid = sid * 2 + cid
    img = wid // 2
    r0 = (wid % 2) * ROWS_PER_TILE

    zi = jnp.zeros((16,), jnp.int32)

    @plsc.parallel_loop(0, K // 16, unroll=8)
    def _(i):
        hp[pl.ds(i * 16, 16)] = zi

    xbufs, tbufs = (xb0, xb1), (tb0, tb1)
    sxs, sts = (sx0, sx1), (st0, st1)

    def start(c):
        rr = pl.ds(r0 + c * CROWS, CROWS)
        hx = pltpu.async_copy(x_hbm.at[img, 0, rr, :], xbufs[c % 2], sxs[c % 2])
        ht = pltpu.async_copy(t_hbm.at[img, 0, rr, :], tbufs[c % 2], sts[c % 2])
        return hx, ht

    handles = start(0)
    for c in range(NCH):
        prev = handles
        if c + 1 < NCH:
            handles = start(c + 1)
        prev[0].wait()
        prev[1].wait()
        xbuf, tbuf = xbufs[c % 2], tbufs[c % 2]

        @plsc.parallel_loop(0, VECS, unroll=8)
        def _(i):
            r = lax.shift_right_logical(i, 5)
            k = jnp.bitwise_and(i, 31)
            xv = xbuf[r, pl.ds(k * 16, 16)]
            tv = tbuf[r, pl.ds(k * 16, 16)]
            f = SCALE - (xv * SCALE) * (2.0 * tv - 1.0)
            fc = jnp.minimum(f, float(K - 2))
            fi = fc.astype(jnp.int32) + 1
            idx = jnp.where(f > 0.0, fi, 0)
            pk = jnp.left_shift(tv.astype(jnp.int32), 16) + 1
            plsc.addupdate_scatter(hp, [idx], pk)

    pltpu.sync_copy(hp, opk_hbm.at[pl.ds(wid * K, K)])


_phase1 = pl.kernel(
    _sc_hist,
    out_type=jax.ShapeDtypeStruct((NTILES * K,), jnp.int32),
    mesh=plsc.VectorSubcoreMesh(core_axis_name="c", subcore_axis_name="s"),
    compiler_params=pltpu.CompilerParams(needs_layout_passes=False),
    scratch_types=[
        pltpu.VMEM((CROWS, 512), jnp.float32),
        pltpu.VMEM((CROWS, 512), jnp.float32),
        pltpu.VMEM((CROWS, 512), jnp.float32),
        pltpu.VMEM((CROWS, 512), jnp.float32),
        pltpu.VMEM((K,), jnp.int32),
        pltpu.SemaphoreType.DMA,
        pltpu.SemaphoreType.DMA,
        pltpu.SemaphoreType.DMA,
        pltpu.SemaphoreType.DMA,
    ],
)


def _tc_finish(hp_ref, o_ref):
    pk = hp_ref[:, 0] + hp_ref[:, 1]
    n3 = jnp.bitwise_and(pk, 0xFFFF).astype(jnp.float32)
    s3 = jnp.right_shift(pk, 16).astype(jnp.float32)

    br = lax.broadcasted_iota(jnp.int32, (KR, KC), 0)
    bc = lax.broadcasted_iota(jnp.int32, (KR, KC), 1)
    bidx = br * KC + bc
    centers = jnp.where(bidx == 0, 0.0,
                        (bidx.astype(jnp.float32) - 0.5) * (1.0 / SCALE))
    w3 = n3 * centers[None]

    r = lax.broadcasted_iota(jnp.int32, (KC, KC), 0)
    c = lax.broadcasted_iota(jnp.int32, (KC, KC), 1)
    upper = (r <= c).astype(jnp.float32)
    strict = (r < c).astype(jnp.float32)

    incl_n = jnp.dot(n3.reshape(B * KR, KC), upper,
                     preferred_element_type=jnp.float32).reshape(B, KR, KC)
    incl_s = jnp.dot(s3.reshape(B * KR, KC), upper,
                     preferred_element_type=jnp.float32).reshape(B, KR, KC)
    rt_n = jnp.sum(n3, axis=2)
    rt_s = jnp.sum(s3, axis=2)
    prev_n = jnp.dot(rt_n, strict, preferred_element_type=jnp.float32)
    prev_s = jnp.dot(rt_s, strict, preferred_element_type=jnp.float32)
    incl_n = incl_n + prev_n[:, :, None]
    incl_s = incl_s + prev_s[:, :, None]

    tot_n = jnp.sum(rt_n, axis=1)[:, None, None]
    g = jnp.sum(rt_s, axis=1)[:, None, None]
    m_above = tot_n - incl_n
    s_above = g - incl_s
    d1 = g + m_above - s_above
    d2 = d1 + n3 - s3
    num = n3 * (g - s_above) + m_above * s3
    dj = jnp.where(
        d1 > 0.0,
        num / jnp.maximum(d1 * d2, 1.0),
        (m_above + n3) / jnp.maximum(d2, 1.0),
    )
    o_ref[0, 0] = jnp.sum(w3 * dj / jnp.maximum(n3, 1.0)) * (1.0 / B)


def _phase2(hpk):
    return pl.pallas_call(
        _tc_finish,
        in_specs=[
            pl.BlockSpec((B, 2, KR, KC), lambda: (0, 0, 0, 0)),
        ],
        out_specs=pl.BlockSpec(
            (1, 1), lambda: (0, 0), memory_space=pltpu.SMEM
        ),
        out_shape=jax.ShapeDtypeStruct((1, 1), jnp.float32),
    )(hpk)


def kernel(input, target):
    hpk = _phase1(input, target)
    out = _phase2(hpk.reshape(B, 2, KR, KC))
    return out[0, 0]


# speedup vs baseline: 87.6133x; 1.1161x over previous
import jax
import jax.numpy as jnp
from jax import lax
from jax.experimental import pallas as pl
from jax.experimental.pallas import tpu as pltpu
from jax.experimental.pallas import tpu_sc as plsc

B = 16
N = 512 * 512
K = 16384
EMAX = 8.0
SCALE = (K - 1) / EMAX
NTILES = 32
ROWS_PER_TILE = 256
CROWS = 16
NCH = ROWS_PER_TILE // CROWS
VECS = CROWS * 512 // 16
KR, KC = 128, 128


def _sc_hist(x_hbm, t_hbm, opk_hbm, xb0, xb1, tb0, tb1, hp,
             sx0, sx1, st0, st1):
    cid = lax.axis_index("c")
    sid = lax.axis_index("s")
    wid = sid * 2 + cid
    img = wid // 2
    r0 = (wid % 2) * ROWS_PER_TILE

    zi = jnp.zeros((16,), jnp.int32)

    @plsc.parallel_loop(0, K // 16, unroll=8)
    def _(i):
        hp[pl.ds(i * 16, 16)] = zi

    xbufs, tbufs = (xb0, xb1), (tb0, tb1)
    sxs, sts = (sx0, sx1), (st0, st1)

    def chunk_copies(c, p):
        rr = pl.ds(r0 + c * CROWS, CROWS)
        return (
            pltpu.make_async_copy(x_hbm.at[img, 0, rr, :], xbufs[p], sxs[p]),
            pltpu.make_async_copy(t_hbm.at[img, 0, rr, :], tbufs[p], sts[p]),
        )

    for p in range(2):
        for h in chunk_copies(p, p):
            h.start()

    def pair_body(j, carry):
        for p in range(2):
            c = 2 * j + p
            for h in chunk_copies(c, p):
                h.wait()
            xbuf, tbuf = xbufs[p], tbufs[p]

            @plsc.parallel_loop(0, VECS, unroll=8)
            def _(i):
                r = lax.shift_right_logical(i, 5)
                k = jnp.bitwise_and(i, 31)
                xv = xbuf[r, pl.ds(k * 16, 16)]
                tv = tbuf[r, pl.ds(k * 16, 16)]
                f = SCALE - (xv * SCALE) * (2.0 * tv - 1.0)
                fc = jnp.minimum(f, float(K - 2))
                idx = jnp.maximum(fc.astype(jnp.int32) + 1, 0)
                pk = jnp.left_shift(tv.astype(jnp.int32), 16) + 1
                plsc.addupdate_scatter(hp, [idx], pk)

            @pl.when(j < NCH // 2 - 1)
            def _():
                for h in chunk_copies(c + 2, p):
                    h.start()
        return carry

    lax.fori_loop(0, NCH // 2, pair_body, 0)

    pltpu.sync_copy(hp, opk_hbm.at[pl.ds(wid * K, K)])


_phase1 = pl.kernel(
    _sc_hist,
    out_type=jax.ShapeDtypeStruct((NTILES * K,), jnp.int32),
    mesh=plsc.VectorSubcoreMesh(core_axis_name="c", subcore_axis_name="s"),
    compiler_params=pltpu.CompilerParams(needs_layout_passes=False),
    scratch_types=[
        pltpu.VMEM((CROWS, 512), jnp.float32),
        pltpu.VMEM((CROWS, 512), jnp.float32),
        pltpu.VMEM((CROWS, 512), jnp.float32),
        pltpu.VMEM((CROWS, 512), jnp.float32),
        pltpu.VMEM((K,), jnp.int32),
        pltpu.SemaphoreType.DMA,
        pltpu.SemaphoreType.DMA,
        pltpu.SemaphoreType.DMA,
        pltpu.SemaphoreType.DMA,
    ],
)


def _tc_finish(hp_ref, o_ref):
    pk = hp_ref[:, 0] + hp_ref[:, 1]
    n3 = jnp.bitwise_and(pk, 0xFFFF).astype(jnp.float32)
    s3 = jnp.right_shift(pk, 16).astype(jnp.float32)

    br = lax.broadcasted_iota(jnp.int32, (KR, KC), 0)
    bc = lax.broadcasted_iota(jnp.int32, (KR, KC), 1)
    bidx = br * KC + bc
    centers = jnp.where(bidx == 0, 0.0,
                        (bidx.astype(jnp.float32) - 0.5) * (1.0 / SCALE))
    w3 = n3 * centers[None]

    r = lax.broadcasted_iota(jnp.int32, (KC, KC), 0)
    c = lax.broadcasted_iota(jnp.int32, (KC, KC), 1)
    upper = (r <= c).astype(jnp.float32)
    strict = (r < c).astype(jnp.float32)

    incl_n = jnp.dot(n3.reshape(B * KR, KC), upper,
                     preferred_element_type=jnp.float32).reshape(B, KR, KC)
    incl_s = jnp.dot(s3.reshape(B * KR, KC), upper,
                     preferred_element_type=jnp.float32).reshape(B, KR, KC)
    rt_n = jnp.sum(n3, axis=2)
    rt_s = jnp.sum(s3, axis=2)
    prev_n = jnp.dot(rt_n, strict, preferred_element_type=jnp.float32)
    prev_s = jnp.dot(rt_s, strict, preferred_element_type=jnp.float32)
    incl_n = incl_n + prev_n[:, :, None]
    incl_s = incl_s + prev_s[:, :, None]

    tot_n = jnp.sum(rt_n, axis=1)[:, None, None]
    g = jnp.sum(rt_s, axis=1)[:, None, None]
    m_above = tot_n - incl_n
    s_above = g - incl_s
    d1 = g + m_above - s_above
    d2 = d1 + n3 - s3
    num = n3 * (g - s_above) + m_above * s3
    dj = jnp.where(
        d1 > 0.0,
        num / jnp.maximum(d1 * d2, 1.0),
        (m_above + n3) / jnp.maximum(d2, 1.0),
    )
    o_ref[0, 0] = jnp.sum(w3 * dj / jnp.maximum(n3, 1.0)) * (1.0 / B)


def _phase2(hpk):
    return pl.pallas_call(
        _tc_finish,
        in_specs=[
            pl.BlockSpec((B, 2, KR, KC), lambda: (0, 0, 0, 0)),
        ],
        out_specs=pl.BlockSpec(
            (1, 1), lambda: (0, 0), memory_space=pltpu.SMEM
        ),
        out_shape=jax.ShapeDtypeStruct((1, 1), jnp.float32),
    )(hpk)


def kernel(input, target):
    hpk = _phase1(input, target)
    out = _phase2(hpk.reshape(B, 2, KR, KC))
    return out[0, 0]
